# baseline (speedup 1.0000x reference)
import functools

import jax
import jax.numpy as jnp
from jax import lax
from jax.experimental import pallas as pl
from jax.experimental.pallas import tpu as pltpu
from jax.experimental.pallas import tpu_sc as plsc

_NUM_CORES = 2
_NUM_SUBCORES = 16
_NW = _NUM_CORES * _NUM_SUBCORES
_L = 16

_D = 64
_W = 128


def _vsqrt(x):
    xi = lax.bitcast_convert_type(x, jnp.int32)
    yi = jnp.int32(0x5F3759DF) - lax.shift_right_logical(xi, 1)
    y = lax.bitcast_convert_type(yi, jnp.float32)
    xh = x * jnp.float32(0.5)
    for _ in range(3):
        y = y * (jnp.float32(1.5) - xh * y * y)
    return x * y


_NCHUNK = 2


def _transe_body(b_per_w, heads_hbm, rels_hbm, tails_hbm, ent_hbm, rel_hbm,
                 out_hbm, hidx_v, ridx_v, tidx_v, hrow_v, rrow_v, trow_v,
                 h_rows, r_rows, t_rows, scores_v, sem):
    wid = lax.axis_index("s") * _NUM_CORES + lax.axis_index("c")
    base = wid * b_per_w
    chunk = b_per_w // _NCHUNK

    pltpu.sync_copy(heads_hbm.at[pl.ds(base, b_per_w)], hidx_v)
    pltpu.sync_copy(rels_hbm.at[pl.ds(base, b_per_w)], ridx_v)
    pltpu.sync_copy(tails_hbm.at[pl.ds(base, b_per_w)], tidx_v)

    def rowify(g, _):
        sl = pl.ds(g * _L, _L)
        hrow_v[sl] = lax.shift_right_logical(hidx_v[sl], 1)
        rrow_v[sl] = lax.shift_right_logical(ridx_v[sl], 1)
        trow_v[sl] = lax.shift_right_logical(tidx_v[sl], 1)
        return _

    lax.fori_loop(0, b_per_w // _L, rowify, 0)

    lanes = lax.iota(jnp.int32, _L)
    half = jnp.int32(_D)
    one = jnp.int32(1)

    for c in range(_NCHUNK):
        csl = pl.ds(c * chunk, chunk)
        ch = pltpu.async_copy(ent_hbm.at[hrow_v.at[csl]], h_rows, sem)
        cr = pltpu.async_copy(rel_hbm.at[rrow_v.at[csl]], r_rows, sem)
        ct = pltpu.async_copy(ent_hbm.at[trow_v.at[csl]], t_rows, sem)
        ch.wait()
        cr.wait()
        ct.wait()

        def group(g, carry):
            sl = pl.ds(c * chunk + g * _L, _L)
            row_idx = g * _L + lanes
            hcol = (hidx_v[sl] & one) * half
            rcol = (ridx_v[sl] & one) * half
            tcol = (tidx_v[sl] & one) * half
            acc = jnp.zeros((_L,), jnp.float32)
            for d in range(_D):
                dd = jnp.int32(d)
                h = plsc.load_gather(h_rows, [row_idx, hcol + dd])
                rl = plsc.load_gather(r_rows, [row_idx, rcol + dd])
                t = plsc.load_gather(t_rows, [row_idx, tcol + dd])
                diff = h + rl - t
                acc = acc + diff * diff
            scores_v[sl] = _vsqrt(acc)
            return carry

        lax.fori_loop(0, chunk // _L, group, 0)

    pltpu.sync_copy(scores_v, out_hbm.at[pl.ds(base, b_per_w)])


def kernel(heads, relations, tails, entity_emb, relation_emb):
    batch = heads.shape[0]
    b_per_w = batch // _NW
    ent2 = jnp.reshape(entity_emb, (entity_emb.shape[0] // 2, _W))
    rel2 = jnp.reshape(relation_emb, (relation_emb.shape[0] // 2, _W))
    mesh = plsc.VectorSubcoreMesh(core_axis_name="c", subcore_axis_name="s")

    k = pl.kernel(
        functools.partial(_transe_body, b_per_w),
        out_type=jax.ShapeDtypeStruct((batch,), jnp.float32),
        mesh=mesh,
        compiler_params=pltpu.CompilerParams(
            needs_layout_passes=False, use_tc_tiling_on_sc=False),
        scratch_types=[
            pltpu.VMEM((b_per_w,), jnp.int32),
            pltpu.VMEM((b_per_w,), jnp.int32),
            pltpu.VMEM((b_per_w,), jnp.int32),
            pltpu.VMEM((b_per_w,), jnp.int32),
            pltpu.VMEM((b_per_w,), jnp.int32),
            pltpu.VMEM((b_per_w,), jnp.int32),
            pltpu.VMEM((b_per_w // _NCHUNK, _W), jnp.float32),
            pltpu.VMEM((b_per_w // _NCHUNK, _W), jnp.float32),
            pltpu.VMEM((b_per_w // _NCHUNK, _W), jnp.float32),
            pltpu.VMEM((b_per_w,), jnp.float32),
            pltpu.SemaphoreType.DMA,
        ],
    )
    return k(heads.astype(jnp.int32), relations.astype(jnp.int32),
             tails.astype(jnp.int32), ent2, rel2)
